# Initial kernel scaffold; baseline (speedup 1.0000x reference)
import functools

import jax
import jax.numpy as jnp
from jax import lax
from jax.experimental import pallas as pl
from jax.experimental.pallas import tpu as pltpu
from jax.experimental.pallas import tpu_sc as plsc

B = 4096
DN = 13
F = 26
V = 100000
D = 32

NC = 2
NS = 16
NW = NC * NS
R = (B * F) // NW
CH = 128
C = R // CH


def _sc_gather(tables_flat, sparse_flat):
    mesh = plsc.VectorSubcoreMesh(core_axis_name="c", subcore_axis_name="s")

    @functools.partial(
        pl.kernel,
        out_type=jax.ShapeDtypeStruct((B * F, D), jnp.float32),
        mesh=mesh,
        scratch_types=[
            pltpu.VMEM((R,), jnp.int32),
            pltpu.VMEM((C, CH), jnp.int32),
            pltpu.VMEM((R, D), jnp.float32),
            pltpu.SemaphoreType.DMA,
        ],
    )
    def k(tab_hbm, sp_hbm, out_hbm, raw_v, idx_v, rows_v, sem):
        wid = lax.axis_index("s") * NC + lax.axis_index("c")
        base = wid * R
        pltpu.sync_copy(sp_hbm.at[pl.ds(base, R)], raw_v)

        def hash_body(i, _):
            s = raw_v[pl.ds(i * 16, 16)]
            pos = (base + i * 16) + lax.iota(jnp.int32, (16,))
            g = (pos % F) * V + (s + 1) % V
            idx_v[i // 8, pl.ds((i % 8) * 16, 16)] = g
            return 0

        lax.fori_loop(0, R // 16, hash_body, 0)

        def fire(j, _):
            pltpu.async_copy(tab_hbm.at[idx_v.at[j]],
                             rows_v.at[pl.ds(j * CH, CH)], sem)
            return 0

        lax.fori_loop(0, C, fire, 0)
        pltpu.make_async_copy(tab_hbm.at[pl.ds(0, R)], rows_v, sem).wait()
        pltpu.sync_copy(rows_v, out_hbm.at[pl.ds(base, R)])

    return k(tables_flat, sparse_flat)


BB = 512


def _mlp_body(dense_ref, emb_ref, mean_ref, std_ref, W1_ref, b1_ref, W2_ref,
              b2_ref, W3_ref, b3_ref, P1a_ref, P1b_ref, pb1_ref, P2_ref,
              pb2_ref, P3_ref, pb3_ref, out_ref):
    x = (dense_ref[...] - mean_ref[...]) / std_ref[...]
    h = jnp.maximum(jnp.dot(x, W1_ref[...], preferred_element_type=jnp.float32)
                    + b1_ref[...], 0.0)
    h = jnp.maximum(jnp.dot(h, W2_ref[...], preferred_element_type=jnp.float32)
                    + b2_ref[...], 0.0)
    dense_out = jnp.dot(h, W3_ref[...], preferred_element_type=jnp.float32) + b3_ref[...]
    h1 = jnp.dot(dense_out, P1a_ref[...], preferred_element_type=jnp.float32)
    h1 = h1 + jnp.dot(emb_ref[...], P1b_ref[...], preferred_element_type=jnp.float32)
    h1 = jnp.maximum(h1 + pb1_ref[...], 0.0)
    h2 = jnp.maximum(jnp.dot(h1, P2_ref[...], preferred_element_type=jnp.float32)
                     + pb2_ref[...], 0.0)
    logit = jnp.sum(h2 * P3_ref[...], axis=1) + pb3_ref[0, 0]
    out_ref[...] = jax.nn.sigmoid(logit)


def _tc_mlp(dense, emb, mean_r, std_r, W1, b1r, W2, b2r, W3, b3r, P1a, P1b,
            pb1r, P2, pb2r, P3r, pb3r):
    grid = (B // BB,)

    def full(shape):
        return pl.BlockSpec(shape, lambda i: (0, 0))

    return pl.pallas_call(
        _mlp_body,
        grid=grid,
        in_specs=[
            pl.BlockSpec((BB, DN), lambda i: (i, 0)),
            pl.BlockSpec((BB, F * D), lambda i: (i, 0)),
            full((1, DN)), full((1, DN)),
            full((DN, 512)), full((1, 512)),
            full((512, 256)), full((1, 256)),
            full((256, D)), full((1, D)),
            full((D, 512)), full((F * D, 512)), full((1, 512)),
            full((512, 256)), full((1, 256)),
            full((1, 256)), full((1, 1)),
        ],
        out_specs=pl.BlockSpec((BB,), lambda i: (i,)),
        out_shape=jax.ShapeDtypeStruct((B,), jnp.float32),
    )(dense, emb, mean_r, std_r, W1, b1r, W2, b2r, W3, b3r, P1a, P1b, pb1r,
      P2, pb2r, P3r, pb3r)


def kernel(dense_features, sparse_features, mean, std, W1, b1, W2, b2, W3, b3,
           tables, P1, pb1, P2, pb2, P3, pb3):
    emb_flat = _sc_gather(tables.reshape(F * V, D),
                          sparse_features.reshape(B * F))
    emb = emb_flat.reshape(B, F * D)
    return _tc_mlp(dense_features, emb, mean.reshape(1, DN), std.reshape(1, DN),
                   W1, b1.reshape(1, 512), W2, b2.reshape(1, 256), W3,
                   b3.reshape(1, D), P1[:D], P1[D:], pb1.reshape(1, 512),
                   P2, pb2.reshape(1, 256), P3.reshape(1, 256),
                   pb3.reshape(1, 1))


# speedup vs baseline: 2.1828x; 2.1828x over previous
import functools

import jax
import jax.numpy as jnp
from jax import lax
from jax.experimental import pallas as pl
from jax.experimental.pallas import tpu as pltpu
from jax.experimental.pallas import tpu_sc as plsc

B = 4096
DN = 13
F = 26
V = 100000
D = 32

NC = 2
NS = 16
NW = NC * NS
R = (B * F) // NW
CH = 128
C = R // CH


def _sc_gather(tables_flat, sparse_flat):
    mesh = plsc.VectorSubcoreMesh(core_axis_name="c", subcore_axis_name="s")

    @functools.partial(
        pl.kernel,
        out_type=jax.ShapeDtypeStruct((B * F, D), jnp.float32),
        mesh=mesh,
        scratch_types=[
            pltpu.VMEM((R,), jnp.int32),
            pltpu.VMEM((C, CH), jnp.int32),
            pltpu.VMEM((R, D), jnp.float32),
            pltpu.SemaphoreType.DMA,
        ],
        compiler_params=pltpu.CompilerParams(use_tc_tiling_on_sc=False),
    )
    def k(tab_hbm, sp_hbm, out_hbm, raw_v, idx_v, rows_v, sem):
        wid = lax.axis_index("s") * NC + lax.axis_index("c")
        base = wid * R
        pltpu.sync_copy(sp_hbm.at[pl.ds(base, R)], raw_v)

        def hash_body(i, _):
            s = raw_v[pl.ds(i * 16, 16)]
            pos = (base + i * 16) + lax.iota(jnp.int32, 16)
            g = (pos % F) * V + (s + 1) % V
            idx_v[i // 8, pl.ds((i % 8) * 16, 16)] = g
            return 0

        lax.fori_loop(0, R // 16, hash_body, 0)

        def fire(j, _):
            pltpu.async_copy(tab_hbm.at[idx_v.at[j]],
                             rows_v.at[pl.ds(j * CH, CH)], sem)
            return 0

        lax.fori_loop(0, C, fire, 0)
        pltpu.make_async_copy(tab_hbm.at[pl.ds(0, R)], rows_v, sem).wait()
        pltpu.sync_copy(rows_v, out_hbm.at[pl.ds(base, R)])

    return k(tables_flat, sparse_flat)


BB = 512


def _mlp_body(dense_ref, emb_ref, mean_ref, std_ref, W1_ref, b1_ref, W2_ref,
              b2_ref, W3_ref, b3_ref, P1a_ref, P1b_ref, pb1_ref, P2_ref,
              pb2_ref, P3_ref, pb3_ref, out_ref):
    x = (dense_ref[...] - mean_ref[...]) / std_ref[...]
    h = jnp.maximum(jnp.dot(x, W1_ref[...], preferred_element_type=jnp.float32)
                    + b1_ref[...], 0.0)
    h = jnp.maximum(jnp.dot(h, W2_ref[...], preferred_element_type=jnp.float32)
                    + b2_ref[...], 0.0)
    dense_out = jnp.dot(h, W3_ref[...], preferred_element_type=jnp.float32) + b3_ref[...]
    h1 = jnp.dot(dense_out, P1a_ref[...], preferred_element_type=jnp.float32)
    h1 = h1 + jnp.dot(emb_ref[...], P1b_ref[...], preferred_element_type=jnp.float32)
    h1 = jnp.maximum(h1 + pb1_ref[...], 0.0)
    h2 = jnp.maximum(jnp.dot(h1, P2_ref[...], preferred_element_type=jnp.float32)
                     + pb2_ref[...], 0.0)
    logit = jnp.sum(h2 * P3_ref[...], axis=1) + pb3_ref[0, 0]
    out_ref[...] = jax.nn.sigmoid(logit)


def _tc_mlp(dense, emb, mean_r, std_r, W1, b1r, W2, b2r, W3, b3r, P1a, P1b,
            pb1r, P2, pb2r, P3r, pb3r):
    grid = (B // BB,)

    def full(shape):
        return pl.BlockSpec(shape, lambda i: (0, 0))

    return pl.pallas_call(
        _mlp_body,
        grid=grid,
        in_specs=[
            pl.BlockSpec((BB, DN), lambda i: (i, 0)),
            pl.BlockSpec((BB, F * D), lambda i: (i, 0)),
            full((1, DN)), full((1, DN)),
            full((DN, 512)), full((1, 512)),
            full((512, 256)), full((1, 256)),
            full((256, D)), full((1, D)),
            full((D, 512)), full((F * D, 512)), full((1, 512)),
            full((512, 256)), full((1, 256)),
            full((1, 256)), full((1, 1)),
        ],
        out_specs=pl.BlockSpec((BB,), lambda i: (i,)),
        out_shape=jax.ShapeDtypeStruct((B,), jnp.float32),
    )(dense, emb, mean_r, std_r, W1, b1r, W2, b2r, W3, b3r, P1a, P1b, pb1r,
      P2, pb2r, P3r, pb3r)


def kernel(dense_features, sparse_features, mean, std, W1, b1, W2, b2, W3, b3,
           tables, P1, pb1, P2, pb2, P3, pb3):
    emb_flat = _sc_gather(tables.reshape(F * V, D),
                          sparse_features.reshape(B * F))
    emb = emb_flat.reshape(B, F * D)
    return _tc_mlp(dense_features, emb, mean.reshape(1, DN), std.reshape(1, DN),
                   W1, b1.reshape(1, 512), W2, b2.reshape(1, 256), W3,
                   b3.reshape(1, D), P1[:D], P1[D:], pb1.reshape(1, 512),
                   P2, pb2.reshape(1, 256), P3.reshape(1, 256),
                   pb3.reshape(1, 1))
